# baseline (speedup 1.0000x reference)
import functools

import jax
import jax.numpy as jnp
from jax import lax
from jax.experimental import pallas as pl
from jax.experimental.pallas import tpu as pltpu
from jax.experimental.pallas import tpu_sc as plsc

NEXP = 8
TAU = 1.0
D = 768
F = 3072
NTOK = 4096
NASSIGN = 2 * NTOK
BM = 128
R_ROUTED = NASSIGN + NEXP * BM
NRBLK = R_ROUTED // BM
NSBLK = NTOK // BM
BT = 512
NW = 32

_SQRT_HALF = 0.7071067811865476


def _k12_body(x_ref, wl_ref, le_ref, b_ref,
              e_out, w_out, rank_ref, offs_ref, gmap_ref,
              stash_i, stash_p, carry_ref):
    i = pl.program_id(0)
    nblk = pl.num_programs(0)
    nrblk = NTOK // BT

    @pl.when(i == 0)
    def _():
        carry_ref[...] = jnp.zeros_like(carry_ref)

    @pl.when(i < nrblk)
    def _():
        x = x_ref[...]
        proj = lax.dot_general(x, wl_ref[...], (((1,), (1,)), ((), ())),
                               preferred_element_type=jnp.float32,
                               precision=lax.Precision.DEFAULT)
        nrm = jnp.sqrt(jnp.sum(proj * proj, axis=1, keepdims=True))
        proj = proj / jnp.maximum(nrm, 1e-12)
        le = le_ref[...]
        ln = jnp.sqrt(jnp.sum(le * le, axis=1, keepdims=True))
        le = le / jnp.maximum(ln, 1e-12)
        logits = lax.dot_general(proj, le, (((1,), (1,)), ((), ())),
                                 preferred_element_type=jnp.float32,
                                 precision=lax.Precision.DEFAULT)
        logits = (logits + b_ref[...]) * (1.0 / TAU)
        m = jnp.max(logits, axis=1, keepdims=True)
        ex = jnp.exp(logits - m)
        p = ex / jnp.sum(ex, axis=1, keepdims=True)
        lane = lax.broadcasted_iota(jnp.int32, p.shape, 1)
        m1 = jnp.max(p, axis=1, keepdims=True)
        i1 = jnp.min(jnp.where(p == m1, lane, NEXP), axis=1, keepdims=True)
        pm = jnp.where(lane == i1, -1.0, p)
        m2 = jnp.max(pm, axis=1, keepdims=True)
        i2 = jnp.min(jnp.where(pm == m2, lane, NEXP), axis=1, keepdims=True)
        e_out[...] = i1
        w_out[...] = m1
        stash_i[pl.ds(i * BT, BT), :] = i2
        stash_p[pl.ds(i * BT, BT), :] = m2

    @pl.when(i >= nrblk)
    def _():
        j = i - nrblk
        e_out[...] = stash_i[pl.ds(j * BT, BT), :]
        w_out[...] = stash_p[pl.ds(j * BT, BT), :]

    e = e_out[...]
    lane = lax.broadcasted_iota(jnp.int32, (BT, NEXP), 1)
    onehot = (e == lane).astype(jnp.float32)
    r = lax.broadcasted_iota(jnp.int32, (BT, BT), 0)
    c = lax.broadcasted_iota(jnp.int32, (BT, BT), 1)
    strict = (c < r).astype(jnp.float32)
    part = lax.dot_general(strict, onehot, (((1,), (0,)), ((), ())),
                           preferred_element_type=jnp.float32)
    carry = carry_ref[0:1, 0:NEXP]
    rank = jnp.sum((part + carry) * onehot, axis=1, keepdims=True)
    rank_ref[...] = rank.astype(jnp.int32)
    counts = carry + jnp.sum(onehot, axis=0, keepdims=True)
    carry_ref[0:1, 0:NEXP] = counts

    @pl.when(i == nblk - 1)
    def _():
        padded = jnp.ceil(counts * (1.0 / BM)) * BM
        ea = lax.broadcasted_iota(jnp.int32, (NEXP, NEXP), 0)
        eb = lax.broadcasted_iota(jnp.int32, (NEXP, NEXP), 1)
        excl = (ea < eb).astype(jnp.float32)
        offs = lax.dot_general(padded, excl, (((1,), (0,)), ((), ())),
                               preferred_element_type=jnp.float32)
        offs_i = offs.astype(jnp.int32)
        offs_ref[...] = jnp.broadcast_to(offs_i, (NEXP, NEXP))
        blk = lax.broadcasted_iota(jnp.int32, (BM, 1), 0)
        start = blk * BM
        cmp = (start >= offs_i).astype(jnp.int32)
        gmap_ref[...] = jnp.sum(cmp, axis=1, keepdims=True) - 1


def _router_dispatch(x2d, wl, le, bias):
    nrblk = NTOK // BT
    n = NASSIGN // BT

    def xmap(i):
        return (jnp.where(i < nrblk, i, nrblk - 1), 0)

    return pl.pallas_call(
        _k12_body,
        grid=(n,),
        in_specs=[
            pl.BlockSpec((BT, D), xmap),
            pl.BlockSpec((D, D), lambda i: (0, 0)),
            pl.BlockSpec((NEXP, D), lambda i: (0, 0)),
            pl.BlockSpec((BT, NEXP), xmap),
        ],
        out_specs=[
            pl.BlockSpec((BT, 1), lambda i: (i, 0)),
            pl.BlockSpec((BT, 1), lambda i: (i, 0)),
            pl.BlockSpec((BT, 1), lambda i: (i, 0)),
            pl.BlockSpec((NEXP, NEXP), lambda i: (0, 0)),
            pl.BlockSpec((BM, 1), lambda i: (0, 0)),
        ],
        out_shape=[
            jax.ShapeDtypeStruct((NASSIGN, 1), jnp.int32),
            jax.ShapeDtypeStruct((NASSIGN, 1), jnp.float32),
            jax.ShapeDtypeStruct((NASSIGN, 1), jnp.int32),
            jax.ShapeDtypeStruct((NEXP, NEXP), jnp.int32),
            jax.ShapeDtypeStruct((BM, 1), jnp.int32),
        ],
        scratch_shapes=[
            pltpu.VMEM((NTOK, 1), jnp.int32),
            pltpu.VMEM((NTOK, 1), jnp.float32),
            pltpu.VMEM((8, 128), jnp.float32),
        ],
    )(x2d, wl, le, bias)


def _sc_dispatch_body(e_h, r_h, w_h, o_h, st_h, wb_h, d_h,
                      e_v, r_v, w_v, o_v, st_v, wb_v, d_v):
    cid = lax.axis_index("c")
    sid = lax.axis_index("s")

    @pl.when(jnp.logical_and(cid == 0, sid == 0))
    def _():
        pltpu.sync_copy(e_h, e_v)
        pltpu.sync_copy(r_h, r_v)
        pltpu.sync_copy(w_h, w_v)
        pltpu.sync_copy(o_h, o_v)
        zi = jnp.zeros((16,), jnp.int32)
        zf = jnp.zeros((16,), jnp.float32)

        def zero(i, carry):
            sl = pl.ds(i * 16, 16)
            st_v[sl] = zi
            wb_v[sl] = zf
            return carry

        lax.fori_loop(0, R_ROUTED // 16, zero, 0)
        lanes = lax.iota(jnp.int32, 16)

        def step(i, carry):
            sl = pl.ds(i * 16, 16)
            e = e_v[sl]
            dest = plsc.load_gather(o_v, [e]) + r_v[sl]
            tok = lax.bitwise_and(lanes + i * 16, NTOK - 1)
            plsc.store_scatter(st_v, [dest], tok)
            plsc.store_scatter(wb_v, [dest], w_v[sl])
            d_v[sl] = dest
            return carry

        lax.fori_loop(0, NASSIGN // 16, step, 0)
        pltpu.sync_copy(st_v, st_h)
        pltpu.sync_copy(wb_v, wb_h)
        pltpu.sync_copy(d_v, d_h)


def _sc_dispatch(e_flat, rank_flat, w_flat, offs16):
    mesh = plsc.VectorSubcoreMesh(core_axis_name="c", subcore_axis_name="s")
    f = functools.partial(
        pl.kernel,
        out_type=[
            jax.ShapeDtypeStruct((R_ROUTED,), jnp.int32),
            jax.ShapeDtypeStruct((R_ROUTED,), jnp.float32),
            jax.ShapeDtypeStruct((NASSIGN,), jnp.int32),
        ],
        mesh=mesh,
        scratch_types=[
            pltpu.VMEM((NASSIGN,), jnp.int32),
            pltpu.VMEM((NASSIGN,), jnp.int32),
            pltpu.VMEM((NASSIGN,), jnp.float32),
            pltpu.VMEM((16,), jnp.int32),
            pltpu.VMEM((R_ROUTED,), jnp.int32),
            pltpu.VMEM((R_ROUTED,), jnp.float32),
            pltpu.VMEM((NASSIGN,), jnp.int32),
        ],
        compiler_params=pltpu.CompilerParams(needs_layout_passes=False),
    )(_sc_dispatch_body)
    return f(e_flat, rank_flat, w_flat, offs16)


_G_ROWS_W = R_ROUTED // NW
_G_CH = 128
_G_CHUNKS = (128, 128, 32)


def _sc_gather_body(x_h, st_h, xs_h, idx_v, rows_a, rows_b, gsa, gsb, wsa, wsb):
    cid = lax.axis_index("c")
    sid = lax.axis_index("s")
    base = (sid * 2 + cid) * _G_ROWS_W
    pltpu.sync_copy(st_h.at[pl.ds(base, _G_ROWS_W)], idx_v)
    rows = (rows_a, rows_b)
    gsem = (gsa, gsb)
    wsem = (wsa, wsb)
    offs = []
    o = 0
    for n in _G_CHUNKS:
        offs.append(o)
        o += n
    nch = len(_G_CHUNKS)
    gcp = [None] * nch
    wcp = [None] * nch

    def start_gather(c):
        cur = c % 2
        n = _G_CHUNKS[c]
        gcp[c] = pltpu.async_copy(
            x_h.at[idx_v.at[pl.ds(offs[c], n)]],
            rows[cur].at[pl.ds(0, n)], gsem[cur])

    start_gather(0)
    for c in range(nch):
        cur = c % 2
        if c + 1 < nch:
            if c >= 1:
                wcp[c - 1].wait()
            start_gather(c + 1)
        gcp[c].wait()
        n = _G_CHUNKS[c]
        wcp[c] = pltpu.async_copy(
            rows[cur].at[pl.ds(0, n)],
            xs_h.at[pl.ds(base + offs[c], n)], wsem[cur])
    wcp[nch - 2].wait()
    wcp[nch - 1].wait()


def _sc_gather(x2d, srctok):
    mesh = plsc.VectorSubcoreMesh(core_axis_name="c", subcore_axis_name="s")
    f = functools.partial(
        pl.kernel,
        out_type=jax.ShapeDtypeStruct((R_ROUTED, D // 2), jnp.int32),
        mesh=mesh,
        scratch_types=[
            pltpu.VMEM((_G_ROWS_W,), jnp.int32),
            pltpu.VMEM((_G_CH, D // 2), jnp.int32),
            pltpu.VMEM((_G_CH, D // 2), jnp.int32),
            pltpu.SemaphoreType.DMA,
            pltpu.SemaphoreType.DMA,
            pltpu.SemaphoreType.DMA,
            pltpu.SemaphoreType.DMA,
        ],
    )(_sc_gather_body)
    return f(x2d, srctok)


def _fca_body(x_ref, w1_ref, b1_ref, w2_ref, b2_ref, y_ref):
    h = lax.dot_general(x_ref[...], w1_ref[0], (((1,), (1,)), ((), ())),
                        preferred_element_type=jnp.float32)
    h = h + b1_ref[0]
    h = 0.5 * h * (1.0 + lax.erf(h * _SQRT_HALF))
    y = lax.dot_general(h, w2_ref[0], (((1,), (1,)), ((), ())),
                        preferred_element_type=jnp.float32)
    y_ref[...] = y + b2_ref[0]


def _fca(x2d, w1, b1r, w2, b2r):
    return pl.pallas_call(
        _fca_body,
        grid=(NSBLK,),
        in_specs=[
            pl.BlockSpec((BM, D), lambda i: (i, 0)),
            pl.BlockSpec((1, F, D), lambda i: (NEXP, 0, 0)),
            pl.BlockSpec((1, 1, F), lambda i: (NEXP, 0, 0)),
            pl.BlockSpec((1, D, F), lambda i: (NEXP, 0, 0)),
            pl.BlockSpec((1, 1, D), lambda i: (NEXP, 0, 0)),
        ],
        out_specs=pl.BlockSpec((BM, D), lambda i: (i, 0)),
        out_shape=jax.ShapeDtypeStruct((NTOK, D), jnp.float32),
        compiler_params=pltpu.CompilerParams(
            dimension_semantics=("arbitrary",)),
    )(x2d, w1, b1r, w2, b2r)


def _fcb_body(g_ref, xs_ref, w1_ref, b1_ref, w2_ref, b2_ref, wv_ref, y_ref):
    x = xs_ref[...].astype(jnp.float32)
    h = lax.dot_general(x, w1_ref[0], (((1,), (1,)), ((), ())),
                        preferred_element_type=jnp.float32)
    h = h + b1_ref[0]
    h = 0.5 * h * (1.0 + lax.erf(h * _SQRT_HALF))
    y = lax.dot_general(h, w2_ref[0], (((1,), (1,)), ((), ())),
                        preferred_element_type=jnp.float32)
    y_ref[...] = (y + b2_ref[0]) * wv_ref[...]


def _fcb(gmap, xs, w1, b1r, w2, b2r, wbuf):
    grid_spec = pltpu.PrefetchScalarGridSpec(
        num_scalar_prefetch=1,
        grid=(NRBLK,),
        in_specs=[
            pl.BlockSpec((BM, D), lambda i, g: (i, 0)),
            pl.BlockSpec((1, F, D), lambda i, g: (g[i], 0, 0)),
            pl.BlockSpec((1, 1, F), lambda i, g: (g[i], 0, 0)),
            pl.BlockSpec((1, D, F), lambda i, g: (g[i], 0, 0)),
            pl.BlockSpec((1, 1, D), lambda i, g: (g[i], 0, 0)),
            pl.BlockSpec((BM, 1), lambda i, g: (i, 0)),
        ],
        out_specs=pl.BlockSpec((BM, D), lambda i, g: (i, 0)),
    )
    return pl.pallas_call(
        _fcb_body,
        grid_spec=grid_spec,
        out_shape=jax.ShapeDtypeStruct((R_ROUTED, D), jnp.float32),
        compiler_params=pltpu.CompilerParams(
            dimension_semantics=("arbitrary",)),
    )(gmap, xs, w1, b1r, w2, b2r, wbuf)


_C_TPW = NTOK // NW
_C_CT = 32


def _acc_rows(av, b0, b1):
    def row(r, carry):
        for k in range(D // 16):
            s = pl.ds(k * 16, 16)
            av[r, s] = av[r, s] + (b0[r, s] + b1[r, s])
        return carry
    lax.fori_loop(0, _C_CT, row, 0)


def _sc_combine_body(ysa_h, ysb_h, d_h, o_h, dv0, dv1, a0, a1, b0, b1,
                     asem, gs0, gs1, ws0, ws1):
    cid = lax.axis_index("c")
    sid = lax.axis_index("s")
    base = (sid * 2 + cid) * _C_TPW
    av = (a0, a1)
    wsem = (ws0, ws1)
    nch = _C_TPW // _C_CT
    wcp = [None] * nch
    for c in range(nch):
        cur = c % 2
        t0 = base + c * _C_CT
        pltpu.sync_copy(d_h.at[pl.ds(t0, _C_CT)], dv0)
        pltpu.sync_copy(d_h.at[pl.ds(NTOK + t0, _C_CT)], dv1)
        if c >= 2:
            wcp[c - 2].wait()
        acp = pltpu.async_copy(ysa_h.at[pl.ds(t0, _C_CT)], av[cur], asem)
        gcp0 = pltpu.async_copy(ysb_h.at[dv0], b0, gs0)
        gcp1 = pltpu.async_copy(ysb_h.at[dv1], b1, gs1)
        acp.wait()
        gcp0.wait()
        gcp1.wait()
        _acc_rows(av[cur], b0, b1)
        wcp[c] = pltpu.async_copy(av[cur], o_h.at[pl.ds(t0, _C_CT)],
                                  wsem[cur])
    wcp[nch - 2].wait()
    wcp[nch - 1].wait()


def _sc_combine(ysa, ysb, dest):
    mesh = plsc.VectorSubcoreMesh(core_axis_name="c", subcore_axis_name="s")
    f = functools.partial(
        pl.kernel,
        out_type=jax.ShapeDtypeStruct((NTOK, D), jnp.float32),
        mesh=mesh,
        scratch_types=[
            pltpu.VMEM((_C_CT,), jnp.int32),
            pltpu.VMEM((_C_CT,), jnp.int32),
            pltpu.VMEM((_C_CT, D), jnp.float32),
            pltpu.VMEM((_C_CT, D), jnp.float32),
            pltpu.VMEM((_C_CT, D), jnp.float32),
            pltpu.VMEM((_C_CT, D), jnp.float32),
            pltpu.SemaphoreType.DMA,
            pltpu.SemaphoreType.DMA,
            pltpu.SemaphoreType.DMA,
            pltpu.SemaphoreType.DMA,
            pltpu.SemaphoreType.DMA,
        ],
    )(_sc_combine_body)
    return f(ysa, ysb, dest)


def kernel(x, Wlinear, learnedE, bias, W1, b1, W2, b2):
    x2d = x.reshape(NTOK, D)
    e_all, w_all, rank, offs88, gmap = _router_dispatch(
        x2d, Wlinear, learnedE, bias)

    offs16 = jnp.pad(offs88[0], (0, 16 - NEXP))
    gmap1 = gmap[:, 0]

    srctok, wbuf, dest = _sc_dispatch(
        e_all.reshape(-1), rank.reshape(-1), w_all.reshape(-1), offs16)

    x16 = x2d.astype(jnp.bfloat16)
    x32 = lax.bitcast_convert_type(x16.reshape(NTOK, D // 2, 2), jnp.int32)
    xs32 = _sc_gather(x32, srctok)
    xs = lax.bitcast_convert_type(xs32, jnp.bfloat16).reshape(R_ROUTED, D)

    b1r = b1.reshape(NEXP + 1, 1, F)
    b2r = b2.reshape(NEXP + 1, 1, D)
    ysa = _fca(x2d, W1, b1r, W2, b2r)
    ysb = _fcb(gmap1, xs, W1, b1r, W2, b2r, wbuf.reshape(R_ROUTED, 1))

    out = _sc_combine(ysa, ysb, dest)
    return out.reshape(x.shape)


# speedup vs baseline: 1.5430x; 1.5430x over previous
import functools

import jax
import jax.numpy as jnp
from jax import lax
from jax.experimental import pallas as pl
from jax.experimental.pallas import tpu as pltpu
from jax.experimental.pallas import tpu_sc as plsc

NEXP = 8
TAU = 1.0
D = 768
F = 3072
NTOK = 4096
NASSIGN = 2 * NTOK
BM = 128
R_ROUTED = NASSIGN + NEXP * BM
NRBLK = R_ROUTED // BM
NSBLK = NTOK // BM
BT = 512
NW = 32

_SQRT_HALF = 0.7071067811865476


def _k12_body(x_ref, wl_ref, le_ref, b_ref,
              e_out, w_out, rank_ref, offs_ref, gmap_ref,
              stash_i, stash_p, carry_ref):
    i = pl.program_id(0)
    nblk = pl.num_programs(0)
    nrblk = NTOK // BT

    @pl.when(i == 0)
    def _():
        carry_ref[...] = jnp.zeros_like(carry_ref)

    @pl.when(i < nrblk)
    def _():
        x = x_ref[...]
        proj = lax.dot_general(x, wl_ref[...], (((1,), (1,)), ((), ())),
                               preferred_element_type=jnp.float32,
                               precision=lax.Precision.DEFAULT)
        nrm = jnp.sqrt(jnp.sum(proj * proj, axis=1, keepdims=True))
        proj = proj / jnp.maximum(nrm, 1e-12)
        le = le_ref[...]
        ln = jnp.sqrt(jnp.sum(le * le, axis=1, keepdims=True))
        le = le / jnp.maximum(ln, 1e-12)
        logits = lax.dot_general(proj, le, (((1,), (1,)), ((), ())),
                                 preferred_element_type=jnp.float32,
                                 precision=lax.Precision.DEFAULT)
        logits = (logits + b_ref[...]) * (1.0 / TAU)
        m = jnp.max(logits, axis=1, keepdims=True)
        ex = jnp.exp(logits - m)
        p = ex / jnp.sum(ex, axis=1, keepdims=True)
        lane = lax.broadcasted_iota(jnp.int32, p.shape, 1)
        m1 = jnp.max(p, axis=1, keepdims=True)
        i1 = jnp.min(jnp.where(p == m1, lane, NEXP), axis=1, keepdims=True)
        pm = jnp.where(lane == i1, -1.0, p)
        m2 = jnp.max(pm, axis=1, keepdims=True)
        i2 = jnp.min(jnp.where(pm == m2, lane, NEXP), axis=1, keepdims=True)
        e_out[...] = i1
        w_out[...] = m1
        stash_i[pl.ds(i * BT, BT), :] = i2
        stash_p[pl.ds(i * BT, BT), :] = m2

    @pl.when(i >= nrblk)
    def _():
        j = i - nrblk
        e_out[...] = stash_i[pl.ds(j * BT, BT), :]
        w_out[...] = stash_p[pl.ds(j * BT, BT), :]

    e = e_out[...]
    lane = lax.broadcasted_iota(jnp.int32, (BT, NEXP), 1)
    onehot = (e == lane).astype(jnp.float32)
    r = lax.broadcasted_iota(jnp.int32, (BT, BT), 0)
    c = lax.broadcasted_iota(jnp.int32, (BT, BT), 1)
    strict = (c < r).astype(jnp.float32)
    part = lax.dot_general(strict, onehot, (((1,), (0,)), ((), ())),
                           preferred_element_type=jnp.float32)
    carry = carry_ref[0:1, 0:NEXP]
    rank = jnp.sum((part + carry) * onehot, axis=1, keepdims=True)
    rank_ref[...] = rank.astype(jnp.int32)
    counts = carry + jnp.sum(onehot, axis=0, keepdims=True)
    carry_ref[0:1, 0:NEXP] = counts

    @pl.when(i == nblk - 1)
    def _():
        padded = jnp.ceil(counts * (1.0 / BM)) * BM
        ea = lax.broadcasted_iota(jnp.int32, (NEXP, NEXP), 0)
        eb = lax.broadcasted_iota(jnp.int32, (NEXP, NEXP), 1)
        excl = (ea < eb).astype(jnp.float32)
        offs = lax.dot_general(padded, excl, (((1,), (0,)), ((), ())),
                               preferred_element_type=jnp.float32)
        offs_i = offs.astype(jnp.int32)
        offs_ref[...] = jnp.broadcast_to(offs_i, (NEXP, NEXP))
        blk = lax.broadcasted_iota(jnp.int32, (BM, 1), 0)
        start = blk * BM
        cmp = (start >= offs_i).astype(jnp.int32)
        gmap_ref[...] = jnp.sum(cmp, axis=1, keepdims=True) - 1


def _router_dispatch(x2d, wl, le, bias):
    nrblk = NTOK // BT
    n = NASSIGN // BT

    def xmap(i):
        return (jnp.where(i < nrblk, i, nrblk - 1), 0)

    return pl.pallas_call(
        _k12_body,
        grid=(n,),
        in_specs=[
            pl.BlockSpec((BT, D), xmap),
            pl.BlockSpec((D, D), lambda i: (0, 0)),
            pl.BlockSpec((NEXP, D), lambda i: (0, 0)),
            pl.BlockSpec((BT, NEXP), xmap),
        ],
        out_specs=[
            pl.BlockSpec((BT, 1), lambda i: (i, 0)),
            pl.BlockSpec((BT, 1), lambda i: (i, 0)),
            pl.BlockSpec((BT, 1), lambda i: (i, 0)),
            pl.BlockSpec((NEXP, NEXP), lambda i: (0, 0)),
            pl.BlockSpec((BM, 1), lambda i: (0, 0)),
        ],
        out_shape=[
            jax.ShapeDtypeStruct((NASSIGN, 1), jnp.int32),
            jax.ShapeDtypeStruct((NASSIGN, 1), jnp.float32),
            jax.ShapeDtypeStruct((NASSIGN, 1), jnp.int32),
            jax.ShapeDtypeStruct((NEXP, NEXP), jnp.int32),
            jax.ShapeDtypeStruct((BM, 1), jnp.int32),
        ],
        scratch_shapes=[
            pltpu.VMEM((NTOK, 1), jnp.int32),
            pltpu.VMEM((NTOK, 1), jnp.float32),
            pltpu.VMEM((8, 128), jnp.float32),
        ],
    )(x2d, wl, le, bias)


_G_ROWS_W = R_ROUTED // NW
_G_CH = 64
_G_CHUNKS = (64, 64, 64, 64, 32)
_A_PW = NASSIGN // NW


def _sc_dispatch_gather_body(e_h, r_h, w_h, o_h, x_h, wb_h, d_h, xs_h,
                             e_v, r_v, w_v, o_v, st_v, wb_v, dv_v,
                             rows_a, rows_b, gsa, gsb, wsa, wsb):
    cid = lax.axis_index("c")
    sid = lax.axis_index("s")
    wid = sid * 2 + cid
    base = wid * _G_ROWS_W
    abase = wid * _A_PW
    pltpu.sync_copy(e_h, e_v)
    pltpu.sync_copy(r_h, r_v)
    pltpu.sync_copy(w_h, w_v)
    pltpu.sync_copy(o_h, o_v)
    zi = jnp.zeros((16,), jnp.int32)
    zf = jnp.zeros((16,), jnp.float32)

    def zero(i, carry):
        sl = pl.ds(i * 16, 16)
        st_v[sl] = zi
        wb_v[sl] = zf
        return carry

    lax.fori_loop(0, _G_ROWS_W // 16, zero, 0)
    lanes = lax.iota(jnp.int32, 16)

    def step(i, carry):
        sl = pl.ds(i * 16, 16)
        e = e_v[sl]
        dest = plsc.load_gather(o_v, [e]) + r_v[sl]
        msk = jnp.logical_and(dest >= base, dest < base + _G_ROWS_W)
        ld = dest - base
        tok = lax.bitwise_and(lanes + i * 16, NTOK - 1)
        plsc.store_scatter(st_v, [ld], tok, mask=msk)
        plsc.store_scatter(wb_v, [ld], w_v[sl], mask=msk)

        @pl.when(i // 16 == wid)
        def _():
            dv_v[pl.ds((i - wid * 16) * 16, 16)] = dest

        return carry

    lax.fori_loop(0, NASSIGN // 16, step, 0)
    pltpu.sync_copy(wb_v, wb_h.at[pl.ds(base, _G_ROWS_W)])
    pltpu.sync_copy(dv_v, d_h.at[pl.ds(abase, _A_PW)])

    rows = (rows_a, rows_b)
    gsem = (gsa, gsb)
    wsem = (wsa, wsb)
    offs = []
    o = 0
    for n in _G_CHUNKS:
        offs.append(o)
        o += n
    nch = len(_G_CHUNKS)
    gcp = [None] * nch
    wcp = [None] * nch

    def start_gather(c):
        cur = c % 2
        n = _G_CHUNKS[c]
        gcp[c] = pltpu.async_copy(
            x_h.at[st_v.at[pl.ds(offs[c], n)]],
            rows[cur].at[pl.ds(0, n)], gsem[cur])

    start_gather(0)
    for c in range(nch):
        cur = c % 2
        if c + 1 < nch:
            if c >= 1:
                wcp[c - 1].wait()
            start_gather(c + 1)
        gcp[c].wait()
        n = _G_CHUNKS[c]
        wcp[c] = pltpu.async_copy(
            rows[cur].at[pl.ds(0, n)],
            xs_h.at[pl.ds(base + offs[c], n)], wsem[cur])
    wcp[nch - 2].wait()
    wcp[nch - 1].wait()


def _sc_dispatch_gather(e_flat, rank_flat, w_flat, offs16, x32):
    mesh = plsc.VectorSubcoreMesh(core_axis_name="c", subcore_axis_name="s")
    f = functools.partial(
        pl.kernel,
        out_type=[
            jax.ShapeDtypeStruct((R_ROUTED,), jnp.float32),
            jax.ShapeDtypeStruct((NASSIGN,), jnp.int32),
            jax.ShapeDtypeStruct((R_ROUTED, D), jnp.float32),
        ],
        mesh=mesh,
        scratch_types=[
            pltpu.VMEM((NASSIGN,), jnp.int32),
            pltpu.VMEM((NASSIGN,), jnp.int32),
            pltpu.VMEM((NASSIGN,), jnp.float32),
            pltpu.VMEM((16,), jnp.int32),
            pltpu.VMEM((_G_ROWS_W,), jnp.int32),
            pltpu.VMEM((_G_ROWS_W,), jnp.float32),
            pltpu.VMEM((_A_PW,), jnp.int32),
            pltpu.VMEM((_G_CH, D), jnp.float32),
            pltpu.VMEM((_G_CH, D), jnp.float32),
            pltpu.SemaphoreType.DMA,
            pltpu.SemaphoreType.DMA,
            pltpu.SemaphoreType.DMA,
            pltpu.SemaphoreType.DMA,
        ],
        compiler_params=pltpu.CompilerParams(needs_layout_passes=False),
    )(_sc_dispatch_gather_body)
    return f(e_flat, rank_flat, w_flat, offs16, x32)


def _fca_body(x_ref, w1_ref, b1_ref, w2_ref, b2_ref, y_ref):
    h = lax.dot_general(x_ref[...], w1_ref[0], (((1,), (1,)), ((), ())),
                        preferred_element_type=jnp.float32)
    h = h + b1_ref[0]
    h = 0.5 * h * (1.0 + lax.erf(h * _SQRT_HALF))
    y = lax.dot_general(h, w2_ref[0], (((1,), (1,)), ((), ())),
                        preferred_element_type=jnp.float32)
    y_ref[...] = y + b2_ref[0]


def _fca(x2d, w1, b1r, w2, b2r):
    return pl.pallas_call(
        _fca_body,
        grid=(NSBLK,),
        in_specs=[
            pl.BlockSpec((BM, D), lambda i: (i, 0)),
            pl.BlockSpec((1, F, D), lambda i: (NEXP, 0, 0)),
            pl.BlockSpec((1, 1, F), lambda i: (NEXP, 0, 0)),
            pl.BlockSpec((1, D, F), lambda i: (NEXP, 0, 0)),
            pl.BlockSpec((1, 1, D), lambda i: (NEXP, 0, 0)),
        ],
        out_specs=pl.BlockSpec((BM, D), lambda i: (i, 0)),
        out_shape=jax.ShapeDtypeStruct((NTOK, D), jnp.float32),
        compiler_params=pltpu.CompilerParams(
            dimension_semantics=("arbitrary",)),
    )(x2d, w1, b1r, w2, b2r)


def _fcb_body(g_ref, xs_ref, w1_ref, b1_ref, w2_ref, b2_ref, wv_ref, y_ref):
    h = lax.dot_general(xs_ref[...], w1_ref[0], (((1,), (1,)), ((), ())),
                        preferred_element_type=jnp.float32)
    h = h + b1_ref[0]
    h = 0.5 * h * (1.0 + lax.erf(h * _SQRT_HALF))
    y = lax.dot_general(h, w2_ref[0], (((1,), (1,)), ((), ())),
                        preferred_element_type=jnp.float32)
    y_ref[...] = (y + b2_ref[0]) * wv_ref[...]


def _fcb(gmap, xs, w1, b1r, w2, b2r, wbuf):
    grid_spec = pltpu.PrefetchScalarGridSpec(
        num_scalar_prefetch=1,
        grid=(NRBLK,),
        in_specs=[
            pl.BlockSpec((BM, D), lambda i, g: (i, 0)),
            pl.BlockSpec((1, F, D), lambda i, g: (g[i], 0, 0)),
            pl.BlockSpec((1, 1, F), lambda i, g: (g[i], 0, 0)),
            pl.BlockSpec((1, D, F), lambda i, g: (g[i], 0, 0)),
            pl.BlockSpec((1, 1, D), lambda i, g: (g[i], 0, 0)),
            pl.BlockSpec((BM, 1), lambda i, g: (i, 0)),
        ],
        out_specs=pl.BlockSpec((BM, D), lambda i, g: (i, 0)),
    )
    return pl.pallas_call(
        _fcb_body,
        grid_spec=grid_spec,
        out_shape=jax.ShapeDtypeStruct((R_ROUTED, D), jnp.float32),
        compiler_params=pltpu.CompilerParams(
            dimension_semantics=("arbitrary",)),
    )(gmap, xs, w1, b1r, w2, b2r, wbuf)


_C_TPW = NTOK // NW
_C_CT = 32


def _acc_rows(av, b0, b1):
    def row(r, carry):
        for k in range(D // 16):
            s = pl.ds(k * 16, 16)
            av[r, s] = av[r, s] + (b0[r, s] + b1[r, s])
        return carry
    lax.fori_loop(0, _C_CT, row, 0)


def _sc_combine_body(ysa_h, ysb_h, d_h, o_h, dv0, dv1, a0, a1, b0, b1,
                     asem, gs0, gs1, ws0, ws1):
    cid = lax.axis_index("c")
    sid = lax.axis_index("s")
    base = (sid * 2 + cid) * _C_TPW
    av = (a0, a1)
    wsem = (ws0, ws1)
    nch = _C_TPW // _C_CT
    wcp = [None] * nch
    for c in range(nch):
        cur = c % 2
        t0 = base + c * _C_CT
        pltpu.sync_copy(d_h.at[pl.ds(t0, _C_CT)], dv0)
        pltpu.sync_copy(d_h.at[pl.ds(NTOK + t0, _C_CT)], dv1)
        if c >= 2:
            wcp[c - 2].wait()
        acp = pltpu.async_copy(ysa_h.at[pl.ds(t0, _C_CT)], av[cur], asem)
        gcp0 = pltpu.async_copy(ysb_h.at[dv0], b0, gs0)
        gcp1 = pltpu.async_copy(ysb_h.at[dv1], b1, gs1)
        acp.wait()
        gcp0.wait()
        gcp1.wait()
        _acc_rows(av[cur], b0, b1)
        wcp[c] = pltpu.async_copy(av[cur], o_h.at[pl.ds(t0, _C_CT)],
                                  wsem[cur])
    wcp[nch - 2].wait()
    wcp[nch - 1].wait()


def _sc_combine(ysa, ysb, dest):
    mesh = plsc.VectorSubcoreMesh(core_axis_name="c", subcore_axis_name="s")
    f = functools.partial(
        pl.kernel,
        out_type=jax.ShapeDtypeStruct((NTOK, D), jnp.float32),
        mesh=mesh,
        scratch_types=[
            pltpu.VMEM((_C_CT,), jnp.int32),
            pltpu.VMEM((_C_CT,), jnp.int32),
            pltpu.VMEM((_C_CT, D), jnp.float32),
            pltpu.VMEM((_C_CT, D), jnp.float32),
            pltpu.VMEM((_C_CT, D), jnp.float32),
            pltpu.VMEM((_C_CT, D), jnp.float32),
            pltpu.SemaphoreType.DMA,
            pltpu.SemaphoreType.DMA,
            pltpu.SemaphoreType.DMA,
            pltpu.SemaphoreType.DMA,
            pltpu.SemaphoreType.DMA,
        ],
    )(_sc_combine_body)
    return f(ysa, ysb, dest)


def kernel(x, Wlinear, learnedE, bias, W1, b1, W2, b2):
    x2d = x.reshape(NTOK, D)
    e_all, w_all, rank, offs88, gmap = _router_dispatch(
        x2d, Wlinear, learnedE, bias)

    offs16 = jnp.pad(offs88[0], (0, 16 - NEXP))
    gmap1 = gmap[:, 0]

    wbuf, dest, xs = _sc_dispatch_gather(
        e_all.reshape(-1), rank.reshape(-1), w_all.reshape(-1), offs16, x2d)

    b1r = b1.reshape(NEXP + 1, 1, F)
    b2r = b2.reshape(NEXP + 1, 1, D)
    ysa = _fca(x2d, W1, b1r, W2, b2r)
    ysb = _fcb(gmap1, xs, W1, b1r, W2, b2r, wbuf.reshape(R_ROUTED, 1))

    out = _sc_combine(ysa, ysb, dest)
    return out.reshape(x.shape)


# speedup vs baseline: 1.5486x; 1.0036x over previous
import functools

import jax
import jax.numpy as jnp
from jax import lax
from jax.experimental import pallas as pl
from jax.experimental.pallas import tpu as pltpu
from jax.experimental.pallas import tpu_sc as plsc

NEXP = 8
TAU = 1.0
D = 768
F = 3072
NTOK = 4096
NASSIGN = 2 * NTOK
BM = 128
R_ROUTED = NASSIGN + NEXP * BM
NRBLK = R_ROUTED // BM
NSBLK = NTOK // BM
BT = 512
NW = 32

_SQRT_HALF = 0.7071067811865476


def _k12_body(x_ref, wl_ref, le_ref, b_ref,
              e_out, w_out, rank_ref, offs_ref, gmap_ref,
              stash_i, stash_p, carry_ref):
    i = pl.program_id(0)
    nblk = pl.num_programs(0)
    nrblk = NTOK // BT

    @pl.when(i == 0)
    def _():
        carry_ref[...] = jnp.zeros_like(carry_ref)

    @pl.when(i < nrblk)
    def _():
        x = x_ref[...]
        proj = lax.dot_general(x, wl_ref[...], (((1,), (1,)), ((), ())),
                               preferred_element_type=jnp.float32,
                               precision=lax.Precision.DEFAULT)
        nrm = jnp.sqrt(jnp.sum(proj * proj, axis=1, keepdims=True))
        proj = proj / jnp.maximum(nrm, 1e-12)
        le = le_ref[...]
        ln = jnp.sqrt(jnp.sum(le * le, axis=1, keepdims=True))
        le = le / jnp.maximum(ln, 1e-12)
        logits = lax.dot_general(proj, le, (((1,), (1,)), ((), ())),
                                 preferred_element_type=jnp.float32,
                                 precision=lax.Precision.DEFAULT)
        logits = (logits + b_ref[...]) * (1.0 / TAU)
        m = jnp.max(logits, axis=1, keepdims=True)
        ex = jnp.exp(logits - m)
        p = ex / jnp.sum(ex, axis=1, keepdims=True)
        lane = lax.broadcasted_iota(jnp.int32, p.shape, 1)
        m1 = jnp.max(p, axis=1, keepdims=True)
        i1 = jnp.min(jnp.where(p == m1, lane, NEXP), axis=1, keepdims=True)
        pm = jnp.where(lane == i1, -1.0, p)
        m2 = jnp.max(pm, axis=1, keepdims=True)
        i2 = jnp.min(jnp.where(pm == m2, lane, NEXP), axis=1, keepdims=True)
        e_out[...] = i1
        w_out[...] = m1
        stash_i[pl.ds(i * BT, BT), :] = i2
        stash_p[pl.ds(i * BT, BT), :] = m2

    @pl.when(i >= nrblk)
    def _():
        j = i - nrblk
        e_out[...] = stash_i[pl.ds(j * BT, BT), :]
        w_out[...] = stash_p[pl.ds(j * BT, BT), :]

    e = e_out[...]
    lane = lax.broadcasted_iota(jnp.int32, (BT, NEXP), 1)
    onehot = (e == lane).astype(jnp.float32)
    r = lax.broadcasted_iota(jnp.int32, (BT, BT), 0)
    c = lax.broadcasted_iota(jnp.int32, (BT, BT), 1)
    strict = (c < r).astype(jnp.float32)
    part = lax.dot_general(strict, onehot, (((1,), (0,)), ((), ())),
                           preferred_element_type=jnp.float32)
    carry = carry_ref[0:1, 0:NEXP]
    rank = jnp.sum((part + carry) * onehot, axis=1, keepdims=True)
    rank_ref[...] = rank.astype(jnp.int32)
    counts = carry + jnp.sum(onehot, axis=0, keepdims=True)
    carry_ref[0:1, 0:NEXP] = counts

    @pl.when(i == nblk - 1)
    def _():
        padded = jnp.ceil(counts * (1.0 / BM)) * BM
        ea = lax.broadcasted_iota(jnp.int32, (NEXP, NEXP), 0)
        eb = lax.broadcasted_iota(jnp.int32, (NEXP, NEXP), 1)
        excl = (ea < eb).astype(jnp.float32)
        offs = lax.dot_general(padded, excl, (((1,), (0,)), ((), ())),
                               preferred_element_type=jnp.float32)
        offs_i = offs.astype(jnp.int32)
        offs_ref[...] = jnp.broadcast_to(offs_i, (NEXP, NEXP))
        blk = lax.broadcasted_iota(jnp.int32, (BM, 1), 0)
        start = blk * BM
        cmp = (start >= offs_i).astype(jnp.int32)
        gmap_ref[...] = jnp.sum(cmp, axis=1, keepdims=True) - 1


def _router_dispatch(x2d, wl, le, bias):
    nrblk = NTOK // BT
    n = NASSIGN // BT

    def xmap(i):
        return (jnp.where(i < nrblk, i, nrblk - 1), 0)

    return pl.pallas_call(
        _k12_body,
        grid=(n,),
        in_specs=[
            pl.BlockSpec((BT, D), xmap),
            pl.BlockSpec((D, D), lambda i: (0, 0)),
            pl.BlockSpec((NEXP, D), lambda i: (0, 0)),
            pl.BlockSpec((BT, NEXP), xmap),
        ],
        out_specs=[
            pl.BlockSpec((BT, 1), lambda i: (i, 0)),
            pl.BlockSpec((BT, 1), lambda i: (i, 0)),
            pl.BlockSpec((BT, 1), lambda i: (i, 0)),
            pl.BlockSpec((NEXP, NEXP), lambda i: (0, 0)),
            pl.BlockSpec((BM, 1), lambda i: (0, 0)),
        ],
        out_shape=[
            jax.ShapeDtypeStruct((NASSIGN, 1), jnp.int32),
            jax.ShapeDtypeStruct((NASSIGN, 1), jnp.float32),
            jax.ShapeDtypeStruct((NASSIGN, 1), jnp.int32),
            jax.ShapeDtypeStruct((NEXP, NEXP), jnp.int32),
            jax.ShapeDtypeStruct((BM, 1), jnp.int32),
        ],
        scratch_shapes=[
            pltpu.VMEM((NTOK, 1), jnp.int32),
            pltpu.VMEM((NTOK, 1), jnp.float32),
            pltpu.VMEM((8, 128), jnp.float32),
        ],
    )(x2d, wl, le, bias)


_G_ROWS_W = R_ROUTED // NW
_G_CH = 64
_G_CHUNKS = (64, 64, 64, 64, 32)
_A_PW = NASSIGN // NW


def _sc_dispatch_gather_body(e_h, r_h, w_h, o_h, x_h, wb_h, d_h, xs_h,
                             e_v, r_v, w_v, o_v, st_v, wb_v, dv_v,
                             rows_a, rows_b, gsa, gsb, wsa, wsb):
    cid = lax.axis_index("c")
    sid = lax.axis_index("s")
    wid = sid * 2 + cid
    base = wid * _G_ROWS_W
    abase = wid * _A_PW
    pltpu.sync_copy(e_h, e_v)
    pltpu.sync_copy(r_h, r_v)
    pltpu.sync_copy(w_h, w_v)
    pltpu.sync_copy(o_h, o_v)
    zi = jnp.zeros((16,), jnp.int32)
    zf = jnp.zeros((16,), jnp.float32)

    def zero(i, carry):
        sl = pl.ds(i * 16, 16)
        st_v[sl] = zi
        wb_v[sl] = zf
        return carry

    lax.fori_loop(0, _G_ROWS_W // 16, zero, 0)
    lanes = lax.iota(jnp.int32, 16)

    def step(i, carry):
        sl = pl.ds(i * 16, 16)
        e = e_v[sl]
        dest = plsc.load_gather(o_v, [e]) + r_v[sl]
        msk = jnp.logical_and(dest >= base, dest < base + _G_ROWS_W)
        ld = dest - base
        tok = lax.bitwise_and(lanes + i * 16, NTOK - 1)
        plsc.store_scatter(st_v, [ld], tok, mask=msk)
        plsc.store_scatter(wb_v, [ld], w_v[sl], mask=msk)

        @pl.when(i // 16 == wid)
        def _():
            dv_v[pl.ds((i - wid * 16) * 16, 16)] = dest

        return carry

    lax.fori_loop(0, NASSIGN // 16, step, 0)
    pltpu.sync_copy(wb_v, wb_h.at[pl.ds(base, _G_ROWS_W)])
    pltpu.sync_copy(dv_v, d_h.at[pl.ds(abase, _A_PW)])

    rows = (rows_a, rows_b)
    gsem = (gsa, gsb)
    wsem = (wsa, wsb)
    offs = []
    o = 0
    for n in _G_CHUNKS:
        offs.append(o)
        o += n
    nch = len(_G_CHUNKS)
    gcp = [None] * nch
    wcp = [None] * nch

    def start_gather(c):
        cur = c % 2
        n = _G_CHUNKS[c]
        gcp[c] = pltpu.async_copy(
            x_h.at[st_v.at[pl.ds(offs[c], n)]],
            rows[cur].at[pl.ds(0, n)], gsem[cur])

    start_gather(0)
    for c in range(nch):
        cur = c % 2
        if c + 1 < nch:
            if c >= 1:
                wcp[c - 1].wait()
            start_gather(c + 1)
        gcp[c].wait()
        n = _G_CHUNKS[c]
        wcp[c] = pltpu.async_copy(
            rows[cur].at[pl.ds(0, n)],
            xs_h.at[pl.ds(base + offs[c], n)], wsem[cur])
    wcp[nch - 2].wait()
    wcp[nch - 1].wait()


def _sc_dispatch_gather(e_flat, rank_flat, w_flat, offs16, x32):
    mesh = plsc.VectorSubcoreMesh(core_axis_name="c", subcore_axis_name="s")
    f = functools.partial(
        pl.kernel,
        out_type=[
            jax.ShapeDtypeStruct((R_ROUTED,), jnp.float32),
            jax.ShapeDtypeStruct((NASSIGN,), jnp.int32),
            jax.ShapeDtypeStruct((R_ROUTED, D), jnp.float32),
        ],
        mesh=mesh,
        scratch_types=[
            pltpu.VMEM((NASSIGN,), jnp.int32),
            pltpu.VMEM((NASSIGN,), jnp.int32),
            pltpu.VMEM((NASSIGN,), jnp.float32),
            pltpu.VMEM((16,), jnp.int32),
            pltpu.VMEM((_G_ROWS_W,), jnp.int32),
            pltpu.VMEM((_G_ROWS_W,), jnp.float32),
            pltpu.VMEM((_A_PW,), jnp.int32),
            pltpu.VMEM((_G_CH, D), jnp.float32),
            pltpu.VMEM((_G_CH, D), jnp.float32),
            pltpu.SemaphoreType.DMA,
            pltpu.SemaphoreType.DMA,
            pltpu.SemaphoreType.DMA,
            pltpu.SemaphoreType.DMA,
        ],
        compiler_params=pltpu.CompilerParams(needs_layout_passes=False),
    )(_sc_dispatch_gather_body)
    return f(e_flat, rank_flat, w_flat, offs16, x32)


def _fca_body(x_ref, w1_ref, b1_ref, w2_ref, b2_ref, y_ref):
    h = lax.dot_general(x_ref[...], w1_ref[0], (((1,), (1,)), ((), ())),
                        preferred_element_type=jnp.float32)
    h = h + b1_ref[0]
    h = 0.5 * h * (1.0 + lax.erf(h * _SQRT_HALF))
    y = lax.dot_general(h, w2_ref[0], (((1,), (1,)), ((), ())),
                        preferred_element_type=jnp.float32)
    y_ref[...] = y + b2_ref[0]


def _fca(x2d, w1, b1r, w2, b2r):
    return pl.pallas_call(
        _fca_body,
        grid=(NSBLK,),
        in_specs=[
            pl.BlockSpec((BM, D), lambda i: (i, 0)),
            pl.BlockSpec((1, F, D), lambda i: (NEXP, 0, 0)),
            pl.BlockSpec((1, 1, F), lambda i: (NEXP, 0, 0)),
            pl.BlockSpec((1, D, F), lambda i: (NEXP, 0, 0)),
            pl.BlockSpec((1, 1, D), lambda i: (NEXP, 0, 0)),
        ],
        out_specs=pl.BlockSpec((BM, D), lambda i: (i, 0)),
        out_shape=jax.ShapeDtypeStruct((NTOK, D), jnp.float32),
        compiler_params=pltpu.CompilerParams(
            dimension_semantics=("arbitrary",)),
    )(x2d, w1, b1r, w2, b2r)


def _fcb_body(g_ref, xs_ref, w1_ref, b1_ref, w2_ref, b2_ref, wv_ref, y_ref):
    h = lax.dot_general(xs_ref[...], w1_ref[0], (((1,), (1,)), ((), ())),
                        preferred_element_type=jnp.float32)
    h = h + b1_ref[0]
    h = 0.5 * h * (1.0 + lax.erf(h * _SQRT_HALF))
    y = lax.dot_general(h, w2_ref[0], (((1,), (1,)), ((), ())),
                        preferred_element_type=jnp.float32)
    y_ref[...] = (y + b2_ref[0]) * wv_ref[...]


def _fcb(gmap, xs, w1, b1r, w2, b2r, wbuf):
    grid_spec = pltpu.PrefetchScalarGridSpec(
        num_scalar_prefetch=1,
        grid=(NRBLK,),
        in_specs=[
            pl.BlockSpec((BM, D), lambda i, g: (i, 0)),
            pl.BlockSpec((1, F, D), lambda i, g: (g[i], 0, 0)),
            pl.BlockSpec((1, 1, F), lambda i, g: (g[i], 0, 0)),
            pl.BlockSpec((1, D, F), lambda i, g: (g[i], 0, 0)),
            pl.BlockSpec((1, 1, D), lambda i, g: (g[i], 0, 0)),
            pl.BlockSpec((BM, 1), lambda i, g: (i, 0)),
        ],
        out_specs=pl.BlockSpec((BM, D), lambda i, g: (i, 0)),
    )
    return pl.pallas_call(
        _fcb_body,
        grid_spec=grid_spec,
        out_shape=jax.ShapeDtypeStruct((R_ROUTED, D), jnp.float32),
        compiler_params=pltpu.CompilerParams(
            dimension_semantics=("arbitrary",)),
    )(gmap, xs, w1, b1r, w2, b2r, wbuf)


_C_TPW = NTOK // NW
_C_CT = 32


def _acc_rows(av, b0, b1):
    def row(r, carry):
        for k in range(D // 16):
            s = pl.ds(k * 16, 16)
            av[r, s] = av[r, s] + (b0[r, s] + b1[r, s])
        return carry
    lax.fori_loop(0, _C_CT, row, 0)


def _sc_combine_body(ysa_h, ysb_h, d_h, o_h, dv0, dv1, a0, a1, b0, b1,
                     asem, gs0, gs1, ws0, ws1):
    cid = lax.axis_index("c")
    sid = lax.axis_index("s")
    base = (sid * 2 + cid) * _C_TPW
    av = (a0, a1)
    wsem = (ws0, ws1)
    nch = _C_TPW // _C_CT
    wcp = [None] * nch
    for c in range(nch):
        cur = c % 2
        t0 = base + c * _C_CT
        pltpu.sync_copy(d_h.at[pl.ds(t0, _C_CT)], dv0)
        pltpu.sync_copy(d_h.at[pl.ds(NTOK + t0, _C_CT)], dv1)
        if c >= 2:
            wcp[c - 2].wait()
        acp = pltpu.async_copy(ysa_h.at[pl.ds(t0, _C_CT)], av[cur], asem)
        gcp0 = pltpu.async_copy(ysb_h.at[dv0], b0, gs0)
        gcp1 = pltpu.async_copy(ysb_h.at[dv1], b1, gs1)
        acp.wait()
        gcp0.wait()
        gcp1.wait()
        _acc_rows(av[cur], b0, b1)
        wcp[c] = pltpu.async_copy(av[cur], o_h.at[pl.ds(t0, _C_CT)],
                                  wsem[cur])
    wcp[nch - 2].wait()
    wcp[nch - 1].wait()


def _sc_combine(ysa, ysb, dest):
    mesh = plsc.VectorSubcoreMesh(core_axis_name="c", subcore_axis_name="s")
    f = functools.partial(
        pl.kernel,
        out_type=jax.ShapeDtypeStruct((NTOK, D), jnp.float32),
        mesh=mesh,
        scratch_types=[
            pltpu.VMEM((_C_CT,), jnp.int32),
            pltpu.VMEM((_C_CT,), jnp.int32),
            pltpu.VMEM((_C_CT, D), jnp.float32),
            pltpu.VMEM((_C_CT, D), jnp.float32),
            pltpu.VMEM((_C_CT, D), jnp.float32),
            pltpu.VMEM((_C_CT, D), jnp.float32),
            pltpu.SemaphoreType.DMA,
            pltpu.SemaphoreType.DMA,
            pltpu.SemaphoreType.DMA,
            pltpu.SemaphoreType.DMA,
            pltpu.SemaphoreType.DMA,
        ],
    )(_sc_combine_body)
    return f(ysa, ysb, dest)


def kernel(x, Wlinear, learnedE, bias, W1, b1, W2, b2):
    x2d = x.reshape(NTOK, D)
    e_all, w_all, rank, offs88, gmap = _router_dispatch(
        x2d, Wlinear, learnedE, bias)

    offs16 = jnp.pad(offs88[0], (0, 16 - NEXP))
    gmap1 = gmap[:, 0]

    b1r = b1.reshape(NEXP + 1, 1, F)
    b2r = b2.reshape(NEXP + 1, 1, D)
    ysa = _fca(x2d, W1, b1r, W2, b2r)

    wbuf, dest, xs = _sc_dispatch_gather(
        e_all.reshape(-1), rank.reshape(-1), w_all.reshape(-1), offs16, x2d)
    ysb = _fcb(gmap1, xs, W1, b1r, W2, b2r, wbuf.reshape(R_ROUTED, 1))

    out = _sc_combine(ysa, ysb, dest)
    return out.reshape(x.shape)


# speedup vs baseline: 1.8855x; 1.2176x over previous
import functools

import jax
import jax.numpy as jnp
from jax import lax
from jax.experimental import pallas as pl
from jax.experimental.pallas import tpu as pltpu
from jax.experimental.pallas import tpu_sc as plsc

NEXP = 8
TAU = 1.0
D = 768
F = 3072
NTOK = 4096
NASSIGN = 2 * NTOK
BM = 256
R_ROUTED = NASSIGN + NEXP * BM
NRBLK = R_ROUTED // BM
NSBLK = NTOK // BM
BT = 512
NW = 32

_SQRT_HALF = 0.7071067811865476


def _k12_body(x_ref, wl_ref, le_ref, b_ref,
              e_out, w_out, rank_ref, offs_ref, gmap_ref,
              stash_i, stash_p, carry_ref):
    i = pl.program_id(0)
    nblk = pl.num_programs(0)
    nrblk = NTOK // BT

    @pl.when(i == 0)
    def _():
        carry_ref[...] = jnp.zeros_like(carry_ref)

    @pl.when(i < nrblk)
    def _():
        x = x_ref[...]
        proj = lax.dot_general(x, wl_ref[...], (((1,), (1,)), ((), ())),
                               preferred_element_type=jnp.float32,
                               precision=lax.Precision.DEFAULT)
        nrm = jnp.sqrt(jnp.sum(proj * proj, axis=1, keepdims=True))
        proj = proj / jnp.maximum(nrm, 1e-12)
        le = le_ref[...]
        ln = jnp.sqrt(jnp.sum(le * le, axis=1, keepdims=True))
        le = le / jnp.maximum(ln, 1e-12)
        logits = lax.dot_general(proj, le, (((1,), (1,)), ((), ())),
                                 preferred_element_type=jnp.float32,
                                 precision=lax.Precision.DEFAULT)
        logits = (logits + b_ref[...]) * (1.0 / TAU)
        m = jnp.max(logits, axis=1, keepdims=True)
        ex = jnp.exp(logits - m)
        p = ex / jnp.sum(ex, axis=1, keepdims=True)
        lane = lax.broadcasted_iota(jnp.int32, p.shape, 1)
        m1 = jnp.max(p, axis=1, keepdims=True)
        i1 = jnp.min(jnp.where(p == m1, lane, NEXP), axis=1, keepdims=True)
        pm = jnp.where(lane == i1, -1.0, p)
        m2 = jnp.max(pm, axis=1, keepdims=True)
        i2 = jnp.min(jnp.where(pm == m2, lane, NEXP), axis=1, keepdims=True)
        e_out[...] = i1
        w_out[...] = m1
        stash_i[pl.ds(i * BT, BT), :] = i2
        stash_p[pl.ds(i * BT, BT), :] = m2

    @pl.when(i >= nrblk)
    def _():
        j = i - nrblk
        e_out[...] = stash_i[pl.ds(j * BT, BT), :]
        w_out[...] = stash_p[pl.ds(j * BT, BT), :]

    e = e_out[...]
    lane = lax.broadcasted_iota(jnp.int32, (BT, NEXP), 1)
    onehot = (e == lane).astype(jnp.float32)
    r = lax.broadcasted_iota(jnp.int32, (BT, BT), 0)
    c = lax.broadcasted_iota(jnp.int32, (BT, BT), 1)
    strict = (c < r).astype(jnp.float32)
    part = lax.dot_general(strict, onehot, (((1,), (0,)), ((), ())),
                           preferred_element_type=jnp.float32)
    carry = carry_ref[0:1, 0:NEXP]
    rank = jnp.sum((part + carry) * onehot, axis=1, keepdims=True)
    rank_ref[...] = rank.astype(jnp.int32)
    counts = carry + jnp.sum(onehot, axis=0, keepdims=True)
    carry_ref[0:1, 0:NEXP] = counts

    @pl.when(i == nblk - 1)
    def _():
        padded = jnp.ceil(counts * (1.0 / BM)) * BM
        ea = lax.broadcasted_iota(jnp.int32, (NEXP, NEXP), 0)
        eb = lax.broadcasted_iota(jnp.int32, (NEXP, NEXP), 1)
        excl = (ea < eb).astype(jnp.float32)
        offs = lax.dot_general(padded, excl, (((1,), (0,)), ((), ())),
                               preferred_element_type=jnp.float32)
        offs_i = offs.astype(jnp.int32)
        offs_ref[...] = jnp.broadcast_to(offs_i, (NEXP, NEXP))
        blk = lax.broadcasted_iota(jnp.int32, (BM, 1), 0)
        start = blk * BM
        cmp = (start >= offs_i).astype(jnp.int32)
        gmap_ref[...] = jnp.sum(cmp, axis=1, keepdims=True) - 1


def _router_dispatch(x2d, wl, le, bias):
    nrblk = NTOK // BT
    n = NASSIGN // BT

    def xmap(i):
        return (jnp.where(i < nrblk, i, nrblk - 1), 0)

    return pl.pallas_call(
        _k12_body,
        grid=(n,),
        in_specs=[
            pl.BlockSpec((BT, D), xmap),
            pl.BlockSpec((D, D), lambda i: (0, 0)),
            pl.BlockSpec((NEXP, D), lambda i: (0, 0)),
            pl.BlockSpec((BT, NEXP), xmap),
        ],
        out_specs=[
            pl.BlockSpec((BT, 1), lambda i: (i, 0)),
            pl.BlockSpec((BT, 1), lambda i: (i, 0)),
            pl.BlockSpec((BT, 1), lambda i: (i, 0)),
            pl.BlockSpec((NEXP, NEXP), lambda i: (0, 0)),
            pl.BlockSpec((BM, 1), lambda i: (0, 0)),
        ],
        out_shape=[
            jax.ShapeDtypeStruct((NASSIGN, 1), jnp.int32),
            jax.ShapeDtypeStruct((NASSIGN, 1), jnp.float32),
            jax.ShapeDtypeStruct((NASSIGN, 1), jnp.int32),
            jax.ShapeDtypeStruct((NEXP, NEXP), jnp.int32),
            jax.ShapeDtypeStruct((BM, 1), jnp.int32),
        ],
        scratch_shapes=[
            pltpu.VMEM((NTOK, 1), jnp.int32),
            pltpu.VMEM((NTOK, 1), jnp.float32),
            pltpu.VMEM((8, 128), jnp.float32),
        ],
    )(x2d, wl, le, bias)


_G_ROWS_W = R_ROUTED // NW
_G_CH = 64
_G_CHUNKS = (64, 64, 64, 64, 64)
_A_PW = NASSIGN // NW


def _sc_dispatch_gather_body(e_h, r_h, w_h, o_h, x_h, wb_h, d_h, xs_h,
                             e_v, r_v, w_v, o_v, st_v, wb_v, dv_v,
                             rows_a, rows_b, gsa, gsb, wsa, wsb):
    cid = lax.axis_index("c")
    sid = lax.axis_index("s")
    wid = sid * 2 + cid
    base = wid * _G_ROWS_W
    abase = wid * _A_PW
    pltpu.sync_copy(e_h, e_v)
    pltpu.sync_copy(r_h, r_v)
    pltpu.sync_copy(w_h, w_v)
    pltpu.sync_copy(o_h, o_v)
    zi = jnp.zeros((16,), jnp.int32)
    zf = jnp.zeros((16,), jnp.float32)

    def zero(i, carry):
        sl = pl.ds(i * 16, 16)
        st_v[sl] = zi
        wb_v[sl] = zf
        return carry

    lax.fori_loop(0, _G_ROWS_W // 16, zero, 0)
    lanes = lax.iota(jnp.int32, 16)

    def step(i, carry):
        sl = pl.ds(i * 16, 16)
        e = e_v[sl]
        dest = plsc.load_gather(o_v, [e]) + r_v[sl]
        msk = jnp.logical_and(dest >= base, dest < base + _G_ROWS_W)
        ld = dest - base
        tok = lax.bitwise_and(lanes + i * 16, NTOK - 1)
        plsc.store_scatter(st_v, [ld], tok, mask=msk)
        plsc.store_scatter(wb_v, [ld], w_v[sl], mask=msk)

        @pl.when(i // 16 == wid)
        def _():
            dv_v[pl.ds((i - wid * 16) * 16, 16)] = dest

        return carry

    lax.fori_loop(0, NASSIGN // 16, step, 0)
    pltpu.sync_copy(wb_v, wb_h.at[pl.ds(base, _G_ROWS_W)])
    pltpu.sync_copy(dv_v, d_h.at[pl.ds(abase, _A_PW)])

    rows = (rows_a, rows_b)
    gsem = (gsa, gsb)
    wsem = (wsa, wsb)
    offs = []
    o = 0
    for n in _G_CHUNKS:
        offs.append(o)
        o += n
    nch = len(_G_CHUNKS)
    gcp = [None] * nch
    wcp = [None] * nch

    def start_gather(c):
        cur = c % 2
        n = _G_CHUNKS[c]
        gcp[c] = pltpu.async_copy(
            x_h.at[st_v.at[pl.ds(offs[c], n)]],
            rows[cur].at[pl.ds(0, n)], gsem[cur])

    start_gather(0)
    for c in range(nch):
        cur = c % 2
        if c + 1 < nch:
            if c >= 1:
                wcp[c - 1].wait()
            start_gather(c + 1)
        gcp[c].wait()
        n = _G_CHUNKS[c]
        wcp[c] = pltpu.async_copy(
            rows[cur].at[pl.ds(0, n)],
            xs_h.at[pl.ds(base + offs[c], n)], wsem[cur])
    wcp[nch - 2].wait()
    wcp[nch - 1].wait()


def _sc_dispatch_gather(e_flat, rank_flat, w_flat, offs16, x32):
    mesh = plsc.VectorSubcoreMesh(core_axis_name="c", subcore_axis_name="s")
    f = functools.partial(
        pl.kernel,
        out_type=[
            jax.ShapeDtypeStruct((R_ROUTED,), jnp.float32),
            jax.ShapeDtypeStruct((NASSIGN,), jnp.int32),
            jax.ShapeDtypeStruct((R_ROUTED, D), jnp.float32),
        ],
        mesh=mesh,
        scratch_types=[
            pltpu.VMEM((NASSIGN,), jnp.int32),
            pltpu.VMEM((NASSIGN,), jnp.int32),
            pltpu.VMEM((NASSIGN,), jnp.float32),
            pltpu.VMEM((16,), jnp.int32),
            pltpu.VMEM((_G_ROWS_W,), jnp.int32),
            pltpu.VMEM((_G_ROWS_W,), jnp.float32),
            pltpu.VMEM((_A_PW,), jnp.int32),
            pltpu.VMEM((_G_CH, D), jnp.float32),
            pltpu.VMEM((_G_CH, D), jnp.float32),
            pltpu.SemaphoreType.DMA,
            pltpu.SemaphoreType.DMA,
            pltpu.SemaphoreType.DMA,
            pltpu.SemaphoreType.DMA,
        ],
        compiler_params=pltpu.CompilerParams(needs_layout_passes=False),
    )(_sc_dispatch_gather_body)
    return f(e_flat, rank_flat, w_flat, offs16, x32)


def _fca_body(x_ref, w1_ref, b1_ref, w2_ref, b2_ref, y_ref):
    h = lax.dot_general(x_ref[...], w1_ref[0], (((1,), (1,)), ((), ())),
                        preferred_element_type=jnp.float32)
    h = h + b1_ref[0]
    h = 0.5 * h * (1.0 + lax.erf(h * _SQRT_HALF))
    y = lax.dot_general(h, w2_ref[0], (((1,), (1,)), ((), ())),
                        preferred_element_type=jnp.float32)
    y_ref[...] = y + b2_ref[0]


def _fca(x2d, w1, b1r, w2, b2r):
    return pl.pallas_call(
        _fca_body,
        grid=(NSBLK,),
        in_specs=[
            pl.BlockSpec((BM, D), lambda i: (i, 0)),
            pl.BlockSpec((1, F, D), lambda i: (NEXP, 0, 0)),
            pl.BlockSpec((1, 1, F), lambda i: (NEXP, 0, 0)),
            pl.BlockSpec((1, D, F), lambda i: (NEXP, 0, 0)),
            pl.BlockSpec((1, 1, D), lambda i: (NEXP, 0, 0)),
        ],
        out_specs=pl.BlockSpec((BM, D), lambda i: (i, 0)),
        out_shape=jax.ShapeDtypeStruct((NTOK, D), jnp.float32),
        compiler_params=pltpu.CompilerParams(
            dimension_semantics=("arbitrary",)),
    )(x2d, w1, b1r, w2, b2r)


def _fcb_body(g_ref, xs_ref, w1_ref, b1_ref, w2_ref, b2_ref, wv_ref, y_ref):
    h = lax.dot_general(xs_ref[...], w1_ref[0], (((1,), (1,)), ((), ())),
                        preferred_element_type=jnp.float32)
    h = h + b1_ref[0]
    h = 0.5 * h * (1.0 + lax.erf(h * _SQRT_HALF))
    y = lax.dot_general(h, w2_ref[0], (((1,), (1,)), ((), ())),
                        preferred_element_type=jnp.float32)
    y_ref[...] = (y + b2_ref[0]) * wv_ref[...]


def _fcb(gmap, xs, w1, b1r, w2, b2r, wbuf):
    grid_spec = pltpu.PrefetchScalarGridSpec(
        num_scalar_prefetch=1,
        grid=(NRBLK,),
        in_specs=[
            pl.BlockSpec((BM, D), lambda i, g: (i, 0)),
            pl.BlockSpec((1, F, D), lambda i, g: (g[i], 0, 0)),
            pl.BlockSpec((1, 1, F), lambda i, g: (g[i], 0, 0)),
            pl.BlockSpec((1, D, F), lambda i, g: (g[i], 0, 0)),
            pl.BlockSpec((1, 1, D), lambda i, g: (g[i], 0, 0)),
            pl.BlockSpec((BM, 1), lambda i, g: (i, 0)),
        ],
        out_specs=pl.BlockSpec((BM, D), lambda i, g: (i, 0)),
    )
    return pl.pallas_call(
        _fcb_body,
        grid_spec=grid_spec,
        out_shape=jax.ShapeDtypeStruct((R_ROUTED, D), jnp.float32),
        compiler_params=pltpu.CompilerParams(
            dimension_semantics=("arbitrary",)),
    )(gmap, xs, w1, b1r, w2, b2r, wbuf)


_C_TPW = NTOK // NW
_C_CT = 32


def _acc_rows(av, b0, b1):
    def row(r, carry):
        for k in range(D // 16):
            s = pl.ds(k * 16, 16)
            av[r, s] = av[r, s] + (b0[r, s] + b1[r, s])
        return carry
    lax.fori_loop(0, _C_CT, row, 0)


def _sc_combine_body(ysa_h, ysb_h, d_h, o_h, dv0, dv1, a0, a1, b0, b1,
                     asem, gs0, gs1, ws0, ws1):
    cid = lax.axis_index("c")
    sid = lax.axis_index("s")
    base = (sid * 2 + cid) * _C_TPW
    av = (a0, a1)
    wsem = (ws0, ws1)
    nch = _C_TPW // _C_CT
    wcp = [None] * nch
    for c in range(nch):
        cur = c % 2
        t0 = base + c * _C_CT
        pltpu.sync_copy(d_h.at[pl.ds(t0, _C_CT)], dv0)
        pltpu.sync_copy(d_h.at[pl.ds(NTOK + t0, _C_CT)], dv1)
        if c >= 2:
            wcp[c - 2].wait()
        acp = pltpu.async_copy(ysa_h.at[pl.ds(t0, _C_CT)], av[cur], asem)
        gcp0 = pltpu.async_copy(ysb_h.at[dv0], b0, gs0)
        gcp1 = pltpu.async_copy(ysb_h.at[dv1], b1, gs1)
        acp.wait()
        gcp0.wait()
        gcp1.wait()
        _acc_rows(av[cur], b0, b1)
        wcp[c] = pltpu.async_copy(av[cur], o_h.at[pl.ds(t0, _C_CT)],
                                  wsem[cur])
    wcp[nch - 2].wait()
    wcp[nch - 1].wait()


def _sc_combine(ysa, ysb, dest):
    mesh = plsc.VectorSubcoreMesh(core_axis_name="c", subcore_axis_name="s")
    f = functools.partial(
        pl.kernel,
        out_type=jax.ShapeDtypeStruct((NTOK, D), jnp.float32),
        mesh=mesh,
        scratch_types=[
            pltpu.VMEM((_C_CT,), jnp.int32),
            pltpu.VMEM((_C_CT,), jnp.int32),
            pltpu.VMEM((_C_CT, D), jnp.float32),
            pltpu.VMEM((_C_CT, D), jnp.float32),
            pltpu.VMEM((_C_CT, D), jnp.float32),
            pltpu.VMEM((_C_CT, D), jnp.float32),
            pltpu.SemaphoreType.DMA,
            pltpu.SemaphoreType.DMA,
            pltpu.SemaphoreType.DMA,
            pltpu.SemaphoreType.DMA,
            pltpu.SemaphoreType.DMA,
        ],
    )(_sc_combine_body)
    return f(ysa, ysb, dest)


def kernel(x, Wlinear, learnedE, bias, W1, b1, W2, b2):
    x2d = x.reshape(NTOK, D)
    e_all, w_all, rank, offs88, gmap = _router_dispatch(
        x2d, Wlinear, learnedE, bias)

    offs16 = jnp.pad(offs88[0], (0, 16 - NEXP))
    gmap1 = gmap[:, 0]

    b1r = b1.reshape(NEXP + 1, 1, F)
    b2r = b2.reshape(NEXP + 1, 1, D)
    ysa = _fca(x2d, W1, b1r, W2, b2r)

    wbuf, dest, xs = _sc_dispatch_gather(
        e_all.reshape(-1), rank.reshape(-1), w_all.reshape(-1), offs16, x2d)
    ysb = _fcb(gmap1, xs, W1, b1r, W2, b2r, wbuf.reshape(R_ROUTED, 1))

    out = _sc_combine(ysa, ysb, dest)
    return out.reshape(x.shape)
